# Initial kernel scaffold; baseline (speedup 1.0000x reference)
import jax
import jax.numpy as jnp
from jax.experimental import pallas as pl


def kernel(x, eis, W_self1, W_neigh1, b1, W_self2, W_neigh2, b2, W_ih, W_hh, b_ih, b_hh, W_out, b_out):
    raise NotImplementedError("write your pallas kernel here")


# speedup vs baseline: 3.7861x; 3.7861x over previous
import functools

import jax
import jax.numpy as jnp
from jax import lax
from jax.experimental import pallas as pl
from jax.experimental.pallas import tpu as pltpu
from jax.experimental.pallas import tpu_sc as plsc

N = 10000
E = 320000
HID = 64
ODIM = 32
T = 3

NC = 2
NS = 16
NW = NC * NS
CH = 80
NBI = 25
NCH_A = E // NS // CH
NCH_B = E // NW // CH
EPT = E // NW

F32 = jnp.float32

_SC_PARAMS = pltpu.CompilerParams(use_tc_tiling_on_sc=False)


_RS = 632
_RSL = N - _RS * (NS - 1)


def _per_stripe(s, fn):
    @pl.when(s < NS - 1)
    def _a():
        fn(pl.multiple_of(s * _RS, 8), _RS)

    @pl.when(s == NS - 1)
    def _b():
        fn((NS - 1) * _RS, _RSL)


def _segsum_body(compute_deg, table, srcA, dstA, srcB, dstB, znd, zn, ones_in,
                 agg_out, deg_out,
                 acc_a, acc_b, dg_a, dg_b,
                 idx_s, idx_d, vals, ones_b, sem):
    c = lax.axis_index("c")
    s = lax.axis_index("s")
    wid = c * NS + s
    pairs = ((acc_a, dg_a), (acc_b, dg_b))

    for acc, dg in pairs:
        def _zero(off, size, acc=acc, dg=dg):
            pltpu.sync_copy(znd.at[pl.ds(off, size)], acc.at[pl.ds(off, size)])
            if compute_deg:
                pltpu.sync_copy(zn.at[pl.ds(off, size)], dg.at[pl.ds(off, size)])
        _per_stripe(s, _zero)
    if compute_deg:
        pltpu.sync_copy(ones_in, ones_b)
    plsc.subcore_barrier()

    def _run(acc, dg, src_h, dst_h, pre, nblk):
        def blk(b, carry):
            pltpu.sync_copy(src_h.at[pre + (pl.ds(b * NBI, NBI),)], idx_s)
            pltpu.sync_copy(dst_h.at[pre + (pl.ds(b * NBI, NBI),)], idx_d)

            def chunk(j, carry2):
                pltpu.async_copy(table.at[idx_s.at[j]], vals, sem).wait()
                pltpu.sync_copy(vals, acc.at[idx_d.at[j]], add=True)
                if compute_deg:
                    pltpu.sync_copy(ones_b, dg.at[idx_d.at[j]], add=True)
                return carry2

            lax.fori_loop(0, NBI, chunk, 0)
            return carry

        lax.fori_loop(0, nblk, blk, 0)

    _run(acc_a, dg_a, srcA, dstA, (c, s), NCH_A // NBI)
    _run(acc_b, dg_b, srcB, dstB, (wid,), NCH_B // NBI)

    plsc.subcore_barrier()
    for r, (acc, dg) in enumerate(pairs):
        def _wout(off, size, r=r, acc=acc, dg=dg):
            pltpu.sync_copy(acc.at[pl.ds(off, size)],
                            agg_out.at[c, r, pl.ds(off, size)])
            if compute_deg:
                pltpu.sync_copy(dg.at[pl.ds(off, size)],
                                deg_out.at[c, r, pl.ds(off, size)])
        _per_stripe(s, _wout)


def _make_segsum(compute_deg):
    mesh = plsc.VectorSubcoreMesh(core_axis_name="c", subcore_axis_name="s",
                                  num_cores=NC, num_subcores=NS)
    out_type = [jax.ShapeDtypeStruct((NC, 2, N, HID), F32),
                jax.ShapeDtypeStruct((NC, 2, N), F32)]
    scratch = [
        pltpu.VMEM_SHARED((N, HID), F32),
        pltpu.VMEM_SHARED((N, HID), F32),
        pltpu.VMEM_SHARED((N,), F32),
        pltpu.VMEM_SHARED((N,), F32),
        pltpu.VMEM((NBI, CH), jnp.int32),
        pltpu.VMEM((NBI, CH), jnp.int32),
        pltpu.VMEM((CH, HID), F32),
        pltpu.VMEM((CH,), F32),
        pltpu.SemaphoreType.DMA,
    ]
    return pl.kernel(functools.partial(_segsum_body, compute_deg),
                     out_type=out_type, mesh=mesh, scratch_types=scratch,
                     compiler_params=_SC_PARAMS)


def _linkpred_body(zcat, sidx, didx, logit_out,
                   idx_s, idx_d, srows, drows, lbuf, sem1, sem2):
    c = lax.axis_index("c")
    s = lax.axis_index("s")
    wid = c * NS + s
    iota16 = lax.iota(jnp.int32, 16)

    for g in range(4):
        def blk(b, carry):
            pltpu.sync_copy(sidx.at[g, wid, pl.ds(b * NBI, NBI)], idx_s)
            pltpu.sync_copy(didx.at[g, wid, pl.ds(b * NBI, NBI)], idx_d)

            def chunk(j, carry2):
                cp1 = pltpu.async_copy(zcat.at[idx_s.at[j]], srows, sem1)
                cp2 = pltpu.async_copy(zcat.at[idx_d.at[j]], drows, sem2)
                cp1.wait()
                cp2.wait()
                jj = b * NBI + j
                for eg in range(CH // 16):
                    acc = jnp.zeros((16,), F32)
                    rows = iota16 + eg * 16
                    for k in range(ODIM):
                        cols = jnp.full((16,), k, jnp.int32)
                        sv = plsc.load_gather(srows, [rows, cols])
                        dv = plsc.load_gather(drows, [rows, cols])
                        acc = acc + sv * dv
                    lbuf[pl.ds(jj * CH + eg * 16, 16)] = acc
                return carry2

            lax.fori_loop(0, NBI, chunk, 0)
            return carry

        lax.fori_loop(0, NCH_B // NBI, blk, 0)
        off = pl.multiple_of((g * NW + wid) * EPT, 8)
        pltpu.sync_copy(lbuf, logit_out.at[pl.ds(off, EPT)])


def _make_linkpred():
    mesh = plsc.VectorSubcoreMesh(core_axis_name="c", subcore_axis_name="s",
                                  num_cores=NC, num_subcores=NS)
    out_type = jax.ShapeDtypeStruct((4 * NW * EPT,), F32)
    scratch = [
        pltpu.VMEM((NBI, CH), jnp.int32),
        pltpu.VMEM((NBI, CH), jnp.int32),
        pltpu.VMEM((CH, ODIM), F32),
        pltpu.VMEM((CH, ODIM), F32),
        pltpu.VMEM((EPT,), F32),
        pltpu.SemaphoreType.DMA,
        pltpu.SemaphoreType.DMA,
    ]
    return pl.kernel(_linkpred_body, out_type=out_type, mesh=mesh,
                     scratch_types=scratch,
                     compiler_params=pltpu.CompilerParams(
                         use_tc_tiling_on_sc=False,
                         needs_layout_passes=False))


_RB = 1000


def _proj1_body(x_ref, wn_ref, ws_ref, b1_ref, y1_ref, xs_ref):
    xb = x_ref[...]
    y1_ref[...] = jnp.dot(xb, wn_ref[...], preferred_element_type=F32)
    xs_ref[...] = jnp.dot(xb, ws_ref[...], preferred_element_type=F32) + b1_ref[...]


def _tc_proj1(x, W_neigh1, W_self1, b1):
    nb = N // _RB
    return pl.pallas_call(
        _proj1_body,
        grid=(nb,),
        in_specs=[
            pl.BlockSpec((_RB, 128), lambda i: (i, 0)),
            pl.BlockSpec((128, HID), lambda i: (0, 0)),
            pl.BlockSpec((128, HID), lambda i: (0, 0)),
            pl.BlockSpec((1, HID), lambda i: (0, 0)),
        ],
        out_specs=[
            pl.BlockSpec((_RB, HID), lambda i: (i, 0)),
            pl.BlockSpec((_RB, HID), lambda i: (i, 0)),
        ],
        out_shape=[jax.ShapeDtypeStruct((N, HID), F32),
                   jax.ShapeDtypeStruct((N, HID), F32)],
    )(x, W_neigh1, W_self1, b1.reshape(1, HID))


def _combine3(p0, p1):
    return (p0[0], p0[1] + p1[1], p1[0])


def _mid_body(xs_ref, a0_ref, a1_ref, d0_ref, d1_ref, wn_ref, ws_ref, b2_ref,
              y2_ref, hs_ref, rd_ref):
    aggs = _combine3(a0_ref, a1_ref)
    degs = _combine3(d0_ref, d1_ref)
    xb = xs_ref[...]
    for t in range(T):
        rd = 1.0 / jnp.maximum(degs[t], 1.0)
        rd_ref[t] = rd
        h1 = jnp.maximum(xb + aggs[t] * rd, 0.0)
        y2_ref[t] = jnp.dot(h1, wn_ref[...], preferred_element_type=F32)
        hs_ref[t] = jnp.dot(h1, ws_ref[...], preferred_element_type=F32) + b2_ref[...]


def _tc_mid(xs, ap, dp, W_neigh2, W_self2, b2):
    nb = N // _RB
    pspec = pl.BlockSpec((2, _RB, HID), lambda i: (0, i, 0))
    dspec = pl.BlockSpec((2, _RB, 1), lambda i: (0, i, 0))
    return pl.pallas_call(
        _mid_body,
        grid=(nb,),
        in_specs=[
            pl.BlockSpec((_RB, HID), lambda i: (i, 0)),
            pspec, pspec, dspec, dspec,
            pl.BlockSpec((HID, HID), lambda i: (0, 0)),
            pl.BlockSpec((HID, HID), lambda i: (0, 0)),
            pl.BlockSpec((1, HID), lambda i: (0, 0)),
        ],
        out_specs=[
            pl.BlockSpec((T, _RB, HID), lambda i: (0, i, 0)),
            pl.BlockSpec((T, _RB, HID), lambda i: (0, i, 0)),
            pl.BlockSpec((T, _RB, 1), lambda i: (0, i, 0)),
        ],
        out_shape=[jax.ShapeDtypeStruct((T, N, HID), F32),
                   jax.ShapeDtypeStruct((T, N, HID), F32),
                   jax.ShapeDtypeStruct((T, N, 1), F32)],
    )(xs, ap[0], ap[1], dp[0].reshape(2, N, 1), dp[1].reshape(2, N, 1),
      W_neigh2, W_self2, b2.reshape(1, HID))


def _gru_body(hs_ref, a0_ref, a1_ref, rd_ref,
              wir_ref, wiz_ref, win_ref, whr_ref, whz_ref, whn_ref,
              bir_ref, biz_ref, bin_ref, bhr_ref, bhz_ref, bhn_ref,
              wo_ref, bo_ref, z01_ref, hf_ref):
    aggs = _combine3(a0_ref, a1_ref)
    h = jnp.zeros((_RB, HID), F32)
    for t in range(T):
        xt = hs_ref[t] + aggs[t] * rd_ref[t]
        ir = jnp.dot(xt, wir_ref[...], preferred_element_type=F32) + bir_ref[...]
        iz = jnp.dot(xt, wiz_ref[...], preferred_element_type=F32) + biz_ref[...]
        inn = jnp.dot(xt, win_ref[...], preferred_element_type=F32) + bin_ref[...]
        hr = jnp.dot(h, whr_ref[...], preferred_element_type=F32) + bhr_ref[...]
        hz = jnp.dot(h, whz_ref[...], preferred_element_type=F32) + bhz_ref[...]
        hn = jnp.dot(h, whn_ref[...], preferred_element_type=F32) + bhn_ref[...]
        r = jax.nn.sigmoid(ir + hr)
        z = jax.nn.sigmoid(iz + hz)
        n = jnp.tanh(inn + r * hn)
        h = (1.0 - z) * n + z * h
        if t < 2:
            z01_ref[t] = jnp.dot(h, wo_ref[...], preferred_element_type=F32) + bo_ref[...]
    hf_ref[...] = h


def _tc_gru(hs, ap, rd, W_ih, W_hh, b_ih, b_hh, W_out, b_out):
    nb = N // _RB
    gate_w = [W_ih[:HID].T, W_ih[HID:2 * HID].T, W_ih[2 * HID:].T,
              W_hh[:HID].T, W_hh[HID:2 * HID].T, W_hh[2 * HID:].T]
    gate_b = [b_ih[:HID].reshape(1, HID), b_ih[HID:2 * HID].reshape(1, HID),
              b_ih[2 * HID:].reshape(1, HID), b_hh[:HID].reshape(1, HID),
              b_hh[HID:2 * HID].reshape(1, HID), b_hh[2 * HID:].reshape(1, HID)]
    wspec = pl.BlockSpec((HID, HID), lambda i: (0, 0))
    bspec = pl.BlockSpec((1, HID), lambda i: (0, 0))
    pspec = pl.BlockSpec((2, _RB, HID), lambda i: (0, i, 0))
    return pl.pallas_call(
        _gru_body,
        grid=(nb,),
        in_specs=[
            pl.BlockSpec((T, _RB, HID), lambda i: (0, i, 0)),
            pspec, pspec,
            pl.BlockSpec((T, _RB, 1), lambda i: (0, i, 0)),
            wspec, wspec, wspec, wspec, wspec, wspec,
            bspec, bspec, bspec, bspec, bspec, bspec,
            pl.BlockSpec((HID, ODIM), lambda i: (0, 0)),
            pl.BlockSpec((1, ODIM), lambda i: (0, 0)),
        ],
        out_specs=[
            pl.BlockSpec((2, _RB, ODIM), lambda i: (0, i, 0)),
            pl.BlockSpec((_RB, HID), lambda i: (i, 0)),
        ],
        out_shape=[jax.ShapeDtypeStruct((2, N, ODIM), F32),
                   jax.ShapeDtypeStruct((N, HID), F32)],
    )(hs, ap[0], ap[1], rd, *gate_w, *gate_b, W_out, b_out.reshape(1, ODIM))


_LB = E // 16


def _loss_body(l_ref, o_ref):
    i = pl.program_id(0)
    l = l_ref[...]
    sgn = jnp.where(i < 4, -1.0, 1.0)
    xx = sgn * l
    sp = jnp.maximum(xx, 0.0) + jnp.log1p(jnp.exp(-jnp.abs(xx)))
    ps = jnp.sum(sp) * (1.0 / (4.0 * E))

    @pl.when(i == 0)
    def _init():
        o_ref[...] = jnp.zeros_like(o_ref)

    o_ref[...] += ps


def _tc_loss(logits):
    return pl.pallas_call(
        _loss_body,
        grid=(8,),
        in_specs=[pl.BlockSpec((8, _LB), lambda i: (i, 0))],
        out_specs=pl.BlockSpec((1, 1), lambda i: (0, 0)),
        out_shape=jax.ShapeDtypeStruct((1, 1), F32),
    )(logits)


def _role_split(idx3):
    a = idx3[jnp.array([0, 2])].reshape(2, NS, NCH_A, CH)
    b = idx3[1].reshape(NW, NCH_B, CH)
    return a, b


def kernel(x, eis, W_self1, W_neigh1, b1, W_self2, W_neigh2, b2,
           W_ih, W_hh, b_ih, b_hh, W_out, b_out):
    eis = eis.astype(jnp.int32)
    src = eis[:, 0, :]
    dst = eis[:, 1, :]
    srcA1, srcB1 = _role_split(src)
    dstA, dstB = _role_split(dst)
    toff = (jnp.arange(T, dtype=jnp.int32) * N)[:, None]
    srcA2, srcB2 = _role_split(src + toff)

    neg_key = jax.random.key(12345)
    rnd = []
    for i in range(T - 1):
        k1, k2 = jax.random.split(jax.random.fold_in(neg_key, i))
        rnd.append((jax.random.randint(k1, (E,), 0, N).astype(jnp.int32),
                    jax.random.randint(k2, (E,), 0, N).astype(jnp.int32)))

    znd = jnp.zeros((N, HID), F32)
    zn = jnp.zeros((N,), F32)
    ones_in = jnp.ones((CH,), F32)

    y1, xs = _tc_proj1(x, W_neigh1, W_self1, b1)
    agg1, degp = _make_segsum(True)(y1, srcA1, dstA, srcB1, dstB,
                                    znd, zn, ones_in)

    y2, hs, rd = _tc_mid(xs, agg1, degp, W_neigh2, W_self2, b2)
    agg2, _ = _make_segsum(False)(y2.reshape(T * N, HID), srcA2, dstA,
                                  srcB2, dstB, znd, zn, ones_in)

    z01, hfin = _tc_gru(hs, agg2, rd, W_ih, W_hh, b_ih, b_hh, W_out, b_out)

    zcat = z01.reshape(2 * N, ODIM)
    sidx = jnp.stack([src[1], src[2] + N, rnd[0][0], rnd[1][0] + N])
    didx = jnp.stack([dst[1], dst[2] + N, rnd[0][1], rnd[1][1] + N])
    logits = _make_linkpred()(zcat,
                              sidx.reshape(4, NW, NCH_B, CH),
                              didx.reshape(4, NW, NCH_B, CH))

    loss = _tc_loss(logits.reshape(64, E // 16))[0, 0]
    return (loss, hfin[None])


# speedup vs baseline: 5.0914x; 1.3448x over previous
import functools

import jax
import jax.numpy as jnp
from jax import lax
from jax.experimental import pallas as pl
from jax.experimental.pallas import tpu as pltpu
from jax.experimental.pallas import tpu_sc as plsc

N = 10000
E = 320000
HID = 64
ODIM = 32
T = 3

NC = 2
NS = 16
NW = NC * NS
CH = 80
NBI = 25
NCH_A = E // NS // CH
NCH_B = E // NW // CH
EPT = E // NW

F32 = jnp.float32

_SC_PARAMS = pltpu.CompilerParams(use_tc_tiling_on_sc=False)


_RS = 632
_RSL = N - _RS * (NS - 1)


def _per_stripe(s, fn):
    @pl.when(s < NS - 1)
    def _a():
        fn(pl.multiple_of(s * _RS, 8), _RS)

    @pl.when(s == NS - 1)
    def _b():
        fn((NS - 1) * _RS, _RSL)


def _segsum_body(compute_deg, table, srcA, dstA, srcB, dstB, znd, zn, ones_in,
                 agg_out, deg_out,
                 acc_a, acc_b, dg_a, dg_b,
                 idx_s, idx_d, vals, vals2, ones_b, sem, sem2):
    c = lax.axis_index("c")
    s = lax.axis_index("s")
    wid = c * NS + s
    pairs = ((acc_a, dg_a), (acc_b, dg_b))

    for acc, dg in pairs:
        def _zero(off, size, acc=acc, dg=dg):
            pltpu.sync_copy(znd.at[pl.ds(off, size)], acc.at[pl.ds(off, size)])
            if compute_deg:
                pltpu.sync_copy(zn.at[pl.ds(off, size)], dg.at[pl.ds(off, size)])
        _per_stripe(s, _zero)
    if compute_deg:
        pltpu.sync_copy(ones_in, ones_b)
    plsc.subcore_barrier()

    def _fire(j, buf, sm):
        pltpu.async_copy(table.at[idx_s.at[j]], buf, sm)

    def _wait(buf, sm):
        pltpu.make_async_copy(table.at[idx_s.at[0]], buf, sm).wait()

    def _run(acc, dg, src_h, dst_h, pre, nblk):
        def _scat(j, buf):
            pltpu.sync_copy(buf, acc.at[idx_d.at[j]], add=True)
            if compute_deg:
                pltpu.sync_copy(ones_b, dg.at[idx_d.at[j]], add=True)

        def blk(b, carry):
            pltpu.sync_copy(src_h.at[pre + (pl.ds(b * NBI, NBI),)], idx_s)
            pltpu.sync_copy(dst_h.at[pre + (pl.ds(b * NBI, NBI),)], idx_d)
            _fire(0, vals, sem)

            def pair(q, carry2):
                _wait(vals, sem)
                _fire(2 * q + 1, vals2, sem2)
                _scat(2 * q, vals)
                _wait(vals2, sem2)
                _fire(2 * q + 2, vals, sem)
                _scat(2 * q + 1, vals2)
                return carry2

            lax.fori_loop(0, (NBI - 1) // 2, pair, 0)
            _wait(vals, sem)
            _scat(NBI - 1, vals)
            return carry

        lax.fori_loop(0, nblk, blk, 0)

    _run(acc_a, dg_a, srcA, dstA, (c, s), NCH_A // NBI)
    _run(acc_b, dg_b, srcB, dstB, (wid,), NCH_B // NBI)

    plsc.subcore_barrier()
    for r, (acc, dg) in enumerate(pairs):
        def _wout(off, size, r=r, acc=acc, dg=dg):
            pltpu.sync_copy(acc.at[pl.ds(off, size)],
                            agg_out.at[c, r, pl.ds(off, size)])
            if compute_deg:
                pltpu.sync_copy(dg.at[pl.ds(off, size)],
                                deg_out.at[c, r, pl.ds(off, size)])
        _per_stripe(s, _wout)


def _make_segsum(compute_deg):
    mesh = plsc.VectorSubcoreMesh(core_axis_name="c", subcore_axis_name="s",
                                  num_cores=NC, num_subcores=NS)
    out_type = [jax.ShapeDtypeStruct((NC, 2, N, HID), F32),
                jax.ShapeDtypeStruct((NC, 2, N), F32)]
    scratch = [
        pltpu.VMEM_SHARED((N, HID), F32),
        pltpu.VMEM_SHARED((N, HID), F32),
        pltpu.VMEM_SHARED((N,), F32),
        pltpu.VMEM_SHARED((N,), F32),
        pltpu.VMEM((NBI, CH), jnp.int32),
        pltpu.VMEM((NBI, CH), jnp.int32),
        pltpu.VMEM((CH, HID), F32),
        pltpu.VMEM((CH, HID), F32),
        pltpu.VMEM((CH,), F32),
        pltpu.SemaphoreType.DMA,
        pltpu.SemaphoreType.DMA,
    ]
    return pl.kernel(functools.partial(_segsum_body, compute_deg),
                     out_type=out_type, mesh=mesh, scratch_types=scratch,
                     compiler_params=_SC_PARAMS)


def _linkpred_body(zcat, sidx, didx, logit_out,
                   idx_s, idx_d, srA, drA, srB, drB, lbuf,
                   sA1, sA2, sB1, sB2):
    c = lax.axis_index("c")
    s = lax.axis_index("s")
    wid = c * NS + s
    iota16 = lax.iota(jnp.int32, 16)

    def fire(j, sr, dr, s1, s2):
        pltpu.async_copy(zcat.at[idx_s.at[j]], sr, s1)
        pltpu.async_copy(zcat.at[idx_d.at[j]], dr, s2)

    def wait(sr, dr, s1, s2):
        pltpu.make_async_copy(zcat.at[idx_s.at[0]], sr, s1).wait()
        pltpu.make_async_copy(zcat.at[idx_d.at[0]], dr, s2).wait()

    def compute(j, sr, dr):
        for eg in range(CH // 16):
            acc = jnp.zeros((16,), F32)
            rows = iota16 + eg * 16
            for k in range(ODIM):
                cols = jnp.full((16,), k, jnp.int32)
                acc = acc + plsc.load_gather(sr, [rows, cols]) * \
                    plsc.load_gather(dr, [rows, cols])
            lbuf[pl.ds(j * CH + eg * 16, 16)] = acc

    for g in range(4):
        pltpu.sync_copy(sidx.at[g, wid], idx_s)
        pltpu.sync_copy(didx.at[g, wid], idx_d)
        fire(0, srA, drA, sA1, sA2)

        def pair(p, carry):
            ja = 2 * p
            wait(srA, drA, sA1, sA2)
            fire(ja + 1, srB, drB, sB1, sB2)
            compute(ja, srA, drA)
            wait(srB, drB, sB1, sB2)
            fire(ja + 2, srA, drA, sA1, sA2)
            compute(ja + 1, srB, drB)
            return carry

        lax.fori_loop(0, (NCH_B - 1) // 2, pair, 0)
        wait(srA, drA, sA1, sA2)
        compute(NCH_B - 1, srA, drA)
        off = pl.multiple_of((g * NW + wid) * EPT, 8)
        pltpu.sync_copy(lbuf, logit_out.at[pl.ds(off, EPT)])


def _make_linkpred():
    mesh = plsc.VectorSubcoreMesh(core_axis_name="c", subcore_axis_name="s",
                                  num_cores=NC, num_subcores=NS)
    out_type = jax.ShapeDtypeStruct((4 * NW * EPT,), F32)
    scratch = [
        pltpu.VMEM((NCH_B, CH), jnp.int32),
        pltpu.VMEM((NCH_B, CH), jnp.int32),
        pltpu.VMEM((CH, ODIM), F32),
        pltpu.VMEM((CH, ODIM), F32),
        pltpu.VMEM((CH, ODIM), F32),
        pltpu.VMEM((CH, ODIM), F32),
        pltpu.VMEM((EPT,), F32),
        pltpu.SemaphoreType.DMA,
        pltpu.SemaphoreType.DMA,
        pltpu.SemaphoreType.DMA,
        pltpu.SemaphoreType.DMA,
    ]
    return pl.kernel(_linkpred_body, out_type=out_type, mesh=mesh,
                     scratch_types=scratch,
                     compiler_params=pltpu.CompilerParams(
                         use_tc_tiling_on_sc=False,
                         needs_layout_passes=False))


_RB = 1000


def _proj1_body(x_ref, wn_ref, ws_ref, b1_ref, y1_ref, xs_ref):
    xb = x_ref[...]
    y1_ref[...] = jnp.dot(xb, wn_ref[...], preferred_element_type=F32)
    xs_ref[...] = jnp.dot(xb, ws_ref[...], preferred_element_type=F32) + b1_ref[...]


def _tc_proj1(x, W_neigh1, W_self1, b1):
    nb = N // _RB
    return pl.pallas_call(
        _proj1_body,
        grid=(nb,),
        in_specs=[
            pl.BlockSpec((_RB, 128), lambda i: (i, 0)),
            pl.BlockSpec((128, HID), lambda i: (0, 0)),
            pl.BlockSpec((128, HID), lambda i: (0, 0)),
            pl.BlockSpec((1, HID), lambda i: (0, 0)),
        ],
        out_specs=[
            pl.BlockSpec((_RB, HID), lambda i: (i, 0)),
            pl.BlockSpec((_RB, HID), lambda i: (i, 0)),
        ],
        out_shape=[jax.ShapeDtypeStruct((N, HID), F32),
                   jax.ShapeDtypeStruct((N, HID), F32)],
    )(x, W_neigh1, W_self1, b1.reshape(1, HID))


def _combine3(p0, p1):
    return (p0[0], p0[1] + p1[1], p1[0])


def _mid_body(xs_ref, a0_ref, a1_ref, d0_ref, d1_ref, wn_ref, ws_ref, b2_ref,
              y2_ref, hs_ref, rd_ref):
    aggs = _combine3(a0_ref, a1_ref)
    degs = _combine3(d0_ref, d1_ref)
    xb = xs_ref[...]
    for t in range(T):
        rd = 1.0 / jnp.maximum(degs[t], 1.0)
        rd_ref[t] = rd
        h1 = jnp.maximum(xb + aggs[t] * rd, 0.0)
        y2_ref[t] = jnp.dot(h1, wn_ref[...], preferred_element_type=F32)
        hs_ref[t] = jnp.dot(h1, ws_ref[...], preferred_element_type=F32) + b2_ref[...]


def _tc_mid(xs, ap, dp, W_neigh2, W_self2, b2):
    nb = N // _RB
    pspec = pl.BlockSpec((2, _RB, HID), lambda i: (0, i, 0))
    dspec = pl.BlockSpec((2, _RB, 1), lambda i: (0, i, 0))
    return pl.pallas_call(
        _mid_body,
        grid=(nb,),
        in_specs=[
            pl.BlockSpec((_RB, HID), lambda i: (i, 0)),
            pspec, pspec, dspec, dspec,
            pl.BlockSpec((HID, HID), lambda i: (0, 0)),
            pl.BlockSpec((HID, HID), lambda i: (0, 0)),
            pl.BlockSpec((1, HID), lambda i: (0, 0)),
        ],
        out_specs=[
            pl.BlockSpec((T, _RB, HID), lambda i: (0, i, 0)),
            pl.BlockSpec((T, _RB, HID), lambda i: (0, i, 0)),
            pl.BlockSpec((T, _RB, 1), lambda i: (0, i, 0)),
        ],
        out_shape=[jax.ShapeDtypeStruct((T, N, HID), F32),
                   jax.ShapeDtypeStruct((T, N, HID), F32),
                   jax.ShapeDtypeStruct((T, N, 1), F32)],
    )(xs, ap[0], ap[1], dp[0].reshape(2, N, 1), dp[1].reshape(2, N, 1),
      W_neigh2, W_self2, b2.reshape(1, HID))


def _gru_body(hs_ref, a0_ref, a1_ref, rd_ref,
              wir_ref, wiz_ref, win_ref, whr_ref, whz_ref, whn_ref,
              bir_ref, biz_ref, bin_ref, bhr_ref, bhz_ref, bhn_ref,
              wo_ref, bo_ref, z01_ref, hf_ref):
    aggs = _combine3(a0_ref, a1_ref)
    h = jnp.zeros((_RB, HID), F32)
    for t in range(T):
        xt = hs_ref[t] + aggs[t] * rd_ref[t]
        ir = jnp.dot(xt, wir_ref[...], preferred_element_type=F32) + bir_ref[...]
        iz = jnp.dot(xt, wiz_ref[...], preferred_element_type=F32) + biz_ref[...]
        inn = jnp.dot(xt, win_ref[...], preferred_element_type=F32) + bin_ref[...]
        hr = jnp.dot(h, whr_ref[...], preferred_element_type=F32) + bhr_ref[...]
        hz = jnp.dot(h, whz_ref[...], preferred_element_type=F32) + bhz_ref[...]
        hn = jnp.dot(h, whn_ref[...], preferred_element_type=F32) + bhn_ref[...]
        r = jax.nn.sigmoid(ir + hr)
        z = jax.nn.sigmoid(iz + hz)
        n = jnp.tanh(inn + r * hn)
        h = (1.0 - z) * n + z * h
        if t < 2:
            z01_ref[t] = jnp.dot(h, wo_ref[...], preferred_element_type=F32) + bo_ref[...]
    hf_ref[...] = h


def _tc_gru(hs, ap, rd, W_ih, W_hh, b_ih, b_hh, W_out, b_out):
    nb = N // _RB
    gate_w = [W_ih[:HID].T, W_ih[HID:2 * HID].T, W_ih[2 * HID:].T,
              W_hh[:HID].T, W_hh[HID:2 * HID].T, W_hh[2 * HID:].T]
    gate_b = [b_ih[:HID].reshape(1, HID), b_ih[HID:2 * HID].reshape(1, HID),
              b_ih[2 * HID:].reshape(1, HID), b_hh[:HID].reshape(1, HID),
              b_hh[HID:2 * HID].reshape(1, HID), b_hh[2 * HID:].reshape(1, HID)]
    wspec = pl.BlockSpec((HID, HID), lambda i: (0, 0))
    bspec = pl.BlockSpec((1, HID), lambda i: (0, 0))
    pspec = pl.BlockSpec((2, _RB, HID), lambda i: (0, i, 0))
    return pl.pallas_call(
        _gru_body,
        grid=(nb,),
        in_specs=[
            pl.BlockSpec((T, _RB, HID), lambda i: (0, i, 0)),
            pspec, pspec,
            pl.BlockSpec((T, _RB, 1), lambda i: (0, i, 0)),
            wspec, wspec, wspec, wspec, wspec, wspec,
            bspec, bspec, bspec, bspec, bspec, bspec,
            pl.BlockSpec((HID, ODIM), lambda i: (0, 0)),
            pl.BlockSpec((1, ODIM), lambda i: (0, 0)),
        ],
        out_specs=[
            pl.BlockSpec((2, _RB, ODIM), lambda i: (0, i, 0)),
            pl.BlockSpec((_RB, HID), lambda i: (i, 0)),
        ],
        out_shape=[jax.ShapeDtypeStruct((2, N, ODIM), F32),
                   jax.ShapeDtypeStruct((N, HID), F32)],
    )(hs, ap[0], ap[1], rd, *gate_w, *gate_b, W_out, b_out.reshape(1, ODIM))


_LB = E // 16


def _loss_body(l_ref, o_ref):
    i = pl.program_id(0)
    l = l_ref[...]
    sgn = jnp.where(i < 4, -1.0, 1.0)
    xx = sgn * l
    sp = jnp.maximum(xx, 0.0) + jnp.log1p(jnp.exp(-jnp.abs(xx)))
    ps = jnp.sum(sp) * (1.0 / (4.0 * E))

    @pl.when(i == 0)
    def _init():
        o_ref[...] = jnp.zeros_like(o_ref)

    o_ref[...] += ps


def _tc_loss(logits):
    return pl.pallas_call(
        _loss_body,
        grid=(8,),
        in_specs=[pl.BlockSpec((8, _LB), lambda i: (i, 0))],
        out_specs=pl.BlockSpec((1, 1), lambda i: (0, 0)),
        out_shape=jax.ShapeDtypeStruct((1, 1), F32),
    )(logits)


def _role_split(idx3):
    a = idx3[jnp.array([0, 2])].reshape(2, NS, NCH_A, CH)
    b = idx3[1].reshape(NW, NCH_B, CH)
    return a, b


def kernel(x, eis, W_self1, W_neigh1, b1, W_self2, W_neigh2, b2,
           W_ih, W_hh, b_ih, b_hh, W_out, b_out):
    eis = eis.astype(jnp.int32)
    src = eis[:, 0, :]
    dst = eis[:, 1, :]
    srcA1, srcB1 = _role_split(src)
    dstA, dstB = _role_split(dst)
    toff = (jnp.arange(T, dtype=jnp.int32) * N)[:, None]
    srcA2, srcB2 = _role_split(src + toff)

    neg_key = jax.random.key(12345)
    rnd = []
    for i in range(T - 1):
        k1, k2 = jax.random.split(jax.random.fold_in(neg_key, i))
        rnd.append((jax.random.randint(k1, (E,), 0, N).astype(jnp.int32),
                    jax.random.randint(k2, (E,), 0, N).astype(jnp.int32)))

    znd = jnp.zeros((N, HID), F32)
    zn = jnp.zeros((N,), F32)
    ones_in = jnp.ones((CH,), F32)

    y1, xs = _tc_proj1(x, W_neigh1, W_self1, b1)
    agg1, degp = _make_segsum(True)(y1, srcA1, dstA, srcB1, dstB,
                                    znd, zn, ones_in)

    y2, hs, rd = _tc_mid(xs, agg1, degp, W_neigh2, W_self2, b2)
    agg2, _ = _make_segsum(False)(y2.reshape(T * N, HID), srcA2, dstA,
                                  srcB2, dstB, znd, zn, ones_in)

    z01, hfin = _tc_gru(hs, agg2, rd, W_ih, W_hh, b_ih, b_hh, W_out, b_out)

    zcat = z01.reshape(2 * N, ODIM)
    sidx = jnp.stack([src[1], src[2] + N, rnd[0][0], rnd[1][0] + N])
    didx = jnp.stack([dst[1], dst[2] + N, rnd[0][1], rnd[1][1] + N])
    logits = _make_linkpred()(zcat,
                              sidx.reshape(4, NW, NCH_B, CH),
                              didx.reshape(4, NW, NCH_B, CH))

    loss = _tc_loss(logits.reshape(64, E // 16))[0, 0]
    return (loss, hfin[None])


# speedup vs baseline: 5.9486x; 1.1684x over previous
import functools

import jax
import jax.numpy as jnp
from jax import lax
from jax.experimental import pallas as pl
from jax.experimental.pallas import tpu as pltpu
from jax.experimental.pallas import tpu_sc as plsc

N = 10000
E = 320000
HID = 64
ODIM = 32
T = 3

NC = 2
NS = 16
NW = NC * NS
CH = 80
NBI = 25
NCH_A = E // NS // CH
NCH_B = E // NW // CH
EPT = E // NW

F32 = jnp.float32

_SC_PARAMS = pltpu.CompilerParams(use_tc_tiling_on_sc=False)


_RS = 632
_RSL = N - _RS * (NS - 1)


def _per_stripe(s, fn):
    @pl.when(s < NS - 1)
    def _a():
        fn(pl.multiple_of(s * _RS, 8), _RS)

    @pl.when(s == NS - 1)
    def _b():
        fn((NS - 1) * _RS, _RSL)


GCH = 5
SGW = GCH * CH
GB = 5


def _segsum_body(compute_deg, table, srcA, dstA, srcB, dstB, znd, zn, ones_in,
                 agg_out, deg_out,
                 acc, dg,
                 idx_s, idx_d, valsA, valsB, ones_b, semA, semB):
    c = lax.axis_index("c")
    s = lax.axis_index("s")
    wid = c * NS + s

    def _zero(off, size):
        pltpu.sync_copy(znd.at[pl.ds(off, size)], acc.at[pl.ds(off, size)])
        if compute_deg:
            pltpu.sync_copy(zn.at[pl.ds(off, size)], dg.at[pl.ds(off, size)])

    def _fire(g, buf, sm):
        pltpu.async_copy(table.at[idx_s.at[g]], buf, sm)

    def _wait(buf, sm):
        pltpu.make_async_copy(table.at[idx_s.at[0]], buf, sm).wait()

    def _scat(g, buf):
        pltpu.sync_copy(buf, acc.at[idx_d.at[g]], add=True)
        if compute_deg:
            pltpu.sync_copy(ones_b, dg.at[idx_d.at[g]], add=True)

    def _run(src_h, dst_h, pre, nblk):
        def blk(b, carry):
            pltpu.sync_copy(src_h.at[pre + (pl.ds(b * GB, GB),)], idx_s)
            pltpu.sync_copy(dst_h.at[pre + (pl.ds(b * GB, GB),)], idx_d)
            _fire(0, valsA, semA)
            for g in range(GB):
                buf, sm = (valsA, semA) if g % 2 == 0 else (valsB, semB)
                nbuf, nsm = (valsB, semB) if g % 2 == 0 else (valsA, semA)
                _wait(buf, sm)
                if g + 1 < GB:
                    _fire(g + 1, nbuf, nsm)
                _scat(g, buf)
            return carry

        lax.fori_loop(0, nblk, blk, 0)

    if compute_deg:
        pltpu.sync_copy(ones_in, ones_b)

    for r in range(2):
        _per_stripe(s, _zero)
        plsc.subcore_barrier()
        if r == 0:
            _run(srcA, dstA, (c, s), NCH_A // GCH // GB)
        else:
            _run(srcB, dstB, (wid,), NCH_B // GCH // GB)
        plsc.subcore_barrier()

        def _wout(off, size, r=r):
            pltpu.sync_copy(acc.at[pl.ds(off, size)],
                            agg_out.at[c, r, pl.ds(off, size)])
            if compute_deg:
                pltpu.sync_copy(dg.at[pl.ds(off, size)],
                                deg_out.at[c, r, pl.ds(off, size)])
        _per_stripe(s, _wout)
        plsc.subcore_barrier()


def _make_segsum(compute_deg):
    mesh = plsc.VectorSubcoreMesh(core_axis_name="c", subcore_axis_name="s",
                                  num_cores=NC, num_subcores=NS)
    out_type = [jax.ShapeDtypeStruct((NC, 2, N, HID), F32),
                jax.ShapeDtypeStruct((NC, 2, N), F32)]
    scratch = [
        pltpu.VMEM_SHARED((N, HID), F32),
        pltpu.VMEM_SHARED((N,), F32),
        pltpu.VMEM((GB, SGW), jnp.int32),
        pltpu.VMEM((GB, SGW), jnp.int32),
        pltpu.VMEM((SGW, HID), F32),
        pltpu.VMEM((SGW, HID), F32),
        pltpu.VMEM((SGW,), F32),
        pltpu.SemaphoreType.DMA,
        pltpu.SemaphoreType.DMA,
    ]
    return pl.kernel(functools.partial(_segsum_body, compute_deg),
                     out_type=out_type, mesh=mesh, scratch_types=scratch,
                     compiler_params=_SC_PARAMS)


def _linkpred_body(zcat, sidx, didx, logit_out,
                   idx_s, idx_d, srA, drA, srB, drB, lbuf,
                   sA1, sA2, sB1, sB2):
    c = lax.axis_index("c")
    s = lax.axis_index("s")
    wid = c * NS + s
    iota16 = lax.iota(jnp.int32, 16)

    def fire(j, sr, dr, s1, s2):
        pltpu.async_copy(zcat.at[idx_s.at[j]], sr, s1)
        pltpu.async_copy(zcat.at[idx_d.at[j]], dr, s2)

    def wait(sr, dr, s1, s2):
        pltpu.make_async_copy(zcat.at[idx_s.at[0]], sr, s1).wait()
        pltpu.make_async_copy(zcat.at[idx_d.at[0]], dr, s2).wait()

    def compute(j, sr, dr):
        def egstep(eg, carry):
            acc = jnp.zeros((16,), F32)
            rows = iota16 + eg * 16
            for k in range(ODIM):
                cols = jnp.full((16,), k, jnp.int32)
                acc = acc + plsc.load_gather(sr, [rows, cols]) * \
                    plsc.load_gather(dr, [rows, cols])
            lbuf[pl.ds(j * SGW + eg * 16, 16)] = acc
            return carry

        lax.fori_loop(0, SGW // 16, egstep, 0)

    NSG = NCH_B // GCH
    for g in range(4):
        pltpu.sync_copy(sidx.at[g, wid], idx_s)
        pltpu.sync_copy(didx.at[g, wid], idx_d)
        fire(0, srA, drA, sA1, sA2)

        def pair(p, carry):
            ja = 2 * p
            wait(srA, drA, sA1, sA2)
            fire(ja + 1, srB, drB, sB1, sB2)
            compute(ja, srA, drA)
            wait(srB, drB, sB1, sB2)
            fire(ja + 2, srA, drA, sA1, sA2)
            compute(ja + 1, srB, drB)
            return carry

        lax.fori_loop(0, (NSG - 1) // 2, pair, 0)
        wait(srA, drA, sA1, sA2)
        compute(NSG - 1, srA, drA)
        off = pl.multiple_of((g * NW + wid) * EPT, 8)
        pltpu.sync_copy(lbuf, logit_out.at[pl.ds(off, EPT)])


def _make_linkpred():
    mesh = plsc.VectorSubcoreMesh(core_axis_name="c", subcore_axis_name="s",
                                  num_cores=NC, num_subcores=NS)
    out_type = jax.ShapeDtypeStruct((4 * NW * EPT,), F32)
    scratch = [
        pltpu.VMEM((EPT // SGW, SGW), jnp.int32),
        pltpu.VMEM((EPT // SGW, SGW), jnp.int32),
        pltpu.VMEM((SGW, ODIM), F32),
        pltpu.VMEM((SGW, ODIM), F32),
        pltpu.VMEM((SGW, ODIM), F32),
        pltpu.VMEM((SGW, ODIM), F32),
        pltpu.VMEM((EPT,), F32),
        pltpu.SemaphoreType.DMA,
        pltpu.SemaphoreType.DMA,
        pltpu.SemaphoreType.DMA,
        pltpu.SemaphoreType.DMA,
    ]
    return pl.kernel(_linkpred_body, out_type=out_type, mesh=mesh,
                     scratch_types=scratch,
                     compiler_params=pltpu.CompilerParams(
                         use_tc_tiling_on_sc=False,
                         needs_layout_passes=False))


_RB = 1000


def _proj1_body(x_ref, wn_ref, ws_ref, b1_ref, y1_ref, xs_ref):
    xb = x_ref[...]
    y1_ref[...] = jnp.dot(xb, wn_ref[...], preferred_element_type=F32)
    xs_ref[...] = jnp.dot(xb, ws_ref[...], preferred_element_type=F32) + b1_ref[...]


def _tc_proj1(x, W_neigh1, W_self1, b1):
    nb = N // _RB
    return pl.pallas_call(
        _proj1_body,
        grid=(nb,),
        in_specs=[
            pl.BlockSpec((_RB, 128), lambda i: (i, 0)),
            pl.BlockSpec((128, HID), lambda i: (0, 0)),
            pl.BlockSpec((128, HID), lambda i: (0, 0)),
            pl.BlockSpec((1, HID), lambda i: (0, 0)),
        ],
        out_specs=[
            pl.BlockSpec((_RB, HID), lambda i: (i, 0)),
            pl.BlockSpec((_RB, HID), lambda i: (i, 0)),
        ],
        out_shape=[jax.ShapeDtypeStruct((N, HID), F32),
                   jax.ShapeDtypeStruct((N, HID), F32)],
    )(x, W_neigh1, W_self1, b1.reshape(1, HID))


def _combine3(p0, p1):
    return (p0[0], p0[1] + p1[1], p1[0])


def _mid_body(xs_ref, a0_ref, a1_ref, d0_ref, d1_ref, wn_ref, ws_ref, b2_ref,
              y2_ref, hs_ref, rd_ref):
    aggs = _combine3(a0_ref, a1_ref)
    degs = _combine3(d0_ref, d1_ref)
    xb = xs_ref[...]
    for t in range(T):
        rd = 1.0 / jnp.maximum(degs[t], 1.0)
        rd_ref[t] = rd
        h1 = jnp.maximum(xb + aggs[t] * rd, 0.0)
        y2_ref[t] = jnp.dot(h1, wn_ref[...], preferred_element_type=F32)
        hs_ref[t] = jnp.dot(h1, ws_ref[...], preferred_element_type=F32) + b2_ref[...]


def _tc_mid(xs, ap, dp, W_neigh2, W_self2, b2):
    nb = N // _RB
    pspec = pl.BlockSpec((2, _RB, HID), lambda i: (0, i, 0))
    dspec = pl.BlockSpec((2, _RB, 1), lambda i: (0, i, 0))
    return pl.pallas_call(
        _mid_body,
        grid=(nb,),
        in_specs=[
            pl.BlockSpec((_RB, HID), lambda i: (i, 0)),
            pspec, pspec, dspec, dspec,
            pl.BlockSpec((HID, HID), lambda i: (0, 0)),
            pl.BlockSpec((HID, HID), lambda i: (0, 0)),
            pl.BlockSpec((1, HID), lambda i: (0, 0)),
        ],
        out_specs=[
            pl.BlockSpec((T, _RB, HID), lambda i: (0, i, 0)),
            pl.BlockSpec((T, _RB, HID), lambda i: (0, i, 0)),
            pl.BlockSpec((T, _RB, 1), lambda i: (0, i, 0)),
        ],
        out_shape=[jax.ShapeDtypeStruct((T, N, HID), F32),
                   jax.ShapeDtypeStruct((T, N, HID), F32),
                   jax.ShapeDtypeStruct((T, N, 1), F32)],
    )(xs, ap[0], ap[1], dp[0].reshape(2, N, 1), dp[1].reshape(2, N, 1),
      W_neigh2, W_self2, b2.reshape(1, HID))


def _gru_body(hs_ref, a0_ref, a1_ref, rd_ref,
              wir_ref, wiz_ref, win_ref, whr_ref, whz_ref, whn_ref,
              bir_ref, biz_ref, bin_ref, bhr_ref, bhz_ref, bhn_ref,
              wo_ref, bo_ref, z01_ref, hf_ref):
    aggs = _combine3(a0_ref, a1_ref)
    h = jnp.zeros((_RB, HID), F32)
    for t in range(T):
        xt = hs_ref[t] + aggs[t] * rd_ref[t]
        ir = jnp.dot(xt, wir_ref[...], preferred_element_type=F32) + bir_ref[...]
        iz = jnp.dot(xt, wiz_ref[...], preferred_element_type=F32) + biz_ref[...]
        inn = jnp.dot(xt, win_ref[...], preferred_element_type=F32) + bin_ref[...]
        hr = jnp.dot(h, whr_ref[...], preferred_element_type=F32) + bhr_ref[...]
        hz = jnp.dot(h, whz_ref[...], preferred_element_type=F32) + bhz_ref[...]
        hn = jnp.dot(h, whn_ref[...], preferred_element_type=F32) + bhn_ref[...]
        r = jax.nn.sigmoid(ir + hr)
        z = jax.nn.sigmoid(iz + hz)
        n = jnp.tanh(inn + r * hn)
        h = (1.0 - z) * n + z * h
        if t < 2:
            z01_ref[t] = jnp.dot(h, wo_ref[...], preferred_element_type=F32) + bo_ref[...]
    hf_ref[...] = h


def _tc_gru(hs, ap, rd, W_ih, W_hh, b_ih, b_hh, W_out, b_out):
    nb = N // _RB
    gate_w = [W_ih[:HID].T, W_ih[HID:2 * HID].T, W_ih[2 * HID:].T,
              W_hh[:HID].T, W_hh[HID:2 * HID].T, W_hh[2 * HID:].T]
    gate_b = [b_ih[:HID].reshape(1, HID), b_ih[HID:2 * HID].reshape(1, HID),
              b_ih[2 * HID:].reshape(1, HID), b_hh[:HID].reshape(1, HID),
              b_hh[HID:2 * HID].reshape(1, HID), b_hh[2 * HID:].reshape(1, HID)]
    wspec = pl.BlockSpec((HID, HID), lambda i: (0, 0))
    bspec = pl.BlockSpec((1, HID), lambda i: (0, 0))
    pspec = pl.BlockSpec((2, _RB, HID), lambda i: (0, i, 0))
    return pl.pallas_call(
        _gru_body,
        grid=(nb,),
        in_specs=[
            pl.BlockSpec((T, _RB, HID), lambda i: (0, i, 0)),
            pspec, pspec,
            pl.BlockSpec((T, _RB, 1), lambda i: (0, i, 0)),
            wspec, wspec, wspec, wspec, wspec, wspec,
            bspec, bspec, bspec, bspec, bspec, bspec,
            pl.BlockSpec((HID, ODIM), lambda i: (0, 0)),
            pl.BlockSpec((1, ODIM), lambda i: (0, 0)),
        ],
        out_specs=[
            pl.BlockSpec((2, _RB, ODIM), lambda i: (0, i, 0)),
            pl.BlockSpec((_RB, HID), lambda i: (i, 0)),
        ],
        out_shape=[jax.ShapeDtypeStruct((2, N, ODIM), F32),
                   jax.ShapeDtypeStruct((N, HID), F32)],
    )(hs, ap[0], ap[1], rd, *gate_w, *gate_b, W_out, b_out.reshape(1, ODIM))


_LB = E // 16


def _loss_body(l_ref, o_ref):
    i = pl.program_id(0)
    l = l_ref[...]
    sgn = jnp.where(i < 4, -1.0, 1.0)
    xx = sgn * l
    sp = jnp.maximum(xx, 0.0) + jnp.log1p(jnp.exp(-jnp.abs(xx)))
    ps = jnp.sum(sp) * (1.0 / (4.0 * E))

    @pl.when(i == 0)
    def _init():
        o_ref[...] = jnp.zeros_like(o_ref)

    o_ref[...] += ps


def _tc_loss(logits):
    return pl.pallas_call(
        _loss_body,
        grid=(8,),
        in_specs=[pl.BlockSpec((8, _LB), lambda i: (i, 0))],
        out_specs=pl.BlockSpec((1, 1), lambda i: (0, 0)),
        out_shape=jax.ShapeDtypeStruct((1, 1), F32),
    )(logits)


def _role_split(idx3):
    a = idx3[jnp.array([0, 2])].reshape(2, NS, NCH_A // GCH, SGW)
    b = idx3[1].reshape(NW, NCH_B // GCH, SGW)
    return a, b


def kernel(x, eis, W_self1, W_neigh1, b1, W_self2, W_neigh2, b2,
           W_ih, W_hh, b_ih, b_hh, W_out, b_out):
    eis = eis.astype(jnp.int32)
    src = eis[:, 0, :]
    dst = eis[:, 1, :]
    srcA1, srcB1 = _role_split(src)
    dstA, dstB = _role_split(dst)
    toff = (jnp.arange(T, dtype=jnp.int32) * N)[:, None]
    srcA2, srcB2 = _role_split(src + toff)

    neg_key = jax.random.key(12345)
    rnd = []
    for i in range(T - 1):
        k1, k2 = jax.random.split(jax.random.fold_in(neg_key, i))
        rnd.append((jax.random.randint(k1, (E,), 0, N).astype(jnp.int32),
                    jax.random.randint(k2, (E,), 0, N).astype(jnp.int32)))

    znd = jnp.zeros((N, HID), F32)
    zn = jnp.zeros((N,), F32)
    ones_in = jnp.ones((SGW,), F32)

    y1, xs = _tc_proj1(x, W_neigh1, W_self1, b1)
    agg1, degp = _make_segsum(True)(y1, srcA1, dstA, srcB1, dstB,
                                    znd, zn, ones_in)

    y2, hs, rd = _tc_mid(xs, agg1, degp, W_neigh2, W_self2, b2)
    agg2, _ = _make_segsum(False)(y2.reshape(T * N, HID), srcA2, dstA,
                                  srcB2, dstB, znd, zn, ones_in)

    z01, hfin = _tc_gru(hs, agg2, rd, W_ih, W_hh, b_ih, b_hh, W_out, b_out)

    zcat = z01.reshape(2 * N, ODIM)
    sidx = jnp.stack([src[1], src[2] + N, rnd[0][0], rnd[1][0] + N])
    didx = jnp.stack([dst[1], dst[2] + N, rnd[0][1], rnd[1][1] + N])
    logits = _make_linkpred()(zcat,
                              sidx.reshape(4, NW, NCH_B // GCH, SGW),
                              didx.reshape(4, NW, NCH_B // GCH, SGW))

    loss = _tc_loss(logits.reshape(64, E // 16))[0, 0]
    return (loss, hfin[None])


# speedup vs baseline: 5.9973x; 1.0082x over previous
import functools

import jax
import jax.numpy as jnp
from jax import lax
from jax.experimental import pallas as pl
from jax.experimental.pallas import tpu as pltpu
from jax.experimental.pallas import tpu_sc as plsc

N = 10000
E = 320000
HID = 64
ODIM = 32
T = 3

NC = 2
NS = 16
NW = NC * NS
CH = 80
NBI = 25
NCH_A = E // NS // CH
NCH_B = E // NW // CH
EPT = E // NW

F32 = jnp.float32

_SC_PARAMS = pltpu.CompilerParams(use_tc_tiling_on_sc=False)


_RS = 632
_RSL = N - _RS * (NS - 1)


def _per_stripe(s, fn):
    @pl.when(s < NS - 1)
    def _a():
        fn(pl.multiple_of(s * _RS, 8), _RS)

    @pl.when(s == NS - 1)
    def _b():
        fn((NS - 1) * _RS, _RSL)


GCH = 5
SGW = GCH * CH
GB = 5


def _segsum_body(compute_deg, table, srcA, dstA, srcB, dstB, znd, zn, ones_in,
                 agg_out, deg_out,
                 acc, dg,
                 idx_s, idx_d, valsA, valsB, ones_b, semA, semB):
    c = lax.axis_index("c")
    s = lax.axis_index("s")
    wid = c * NS + s

    def _zero(off, size):
        pltpu.sync_copy(znd.at[pl.ds(off, size)], acc.at[pl.ds(off, size)])
        if compute_deg:
            pltpu.sync_copy(zn.at[pl.ds(off, size)], dg.at[pl.ds(off, size)])

    def _fire(g, buf, sm):
        pltpu.async_copy(table.at[idx_s.at[g]], buf, sm)

    def _wait(buf, sm):
        pltpu.make_async_copy(table.at[idx_s.at[0]], buf, sm).wait()

    def _scat(g, buf):
        pltpu.sync_copy(buf, acc.at[idx_d.at[g]], add=True)
        if compute_deg:
            pltpu.sync_copy(ones_b, dg.at[idx_d.at[g]], add=True)

    def _run(src_h, dst_h, pre, nblk):
        def blk(b, carry):
            pltpu.sync_copy(src_h.at[pre + (pl.ds(b * GB, GB),)], idx_s)
            pltpu.sync_copy(dst_h.at[pre + (pl.ds(b * GB, GB),)], idx_d)
            _fire(0, valsA, semA)
            for g in range(GB):
                buf, sm = (valsA, semA) if g % 2 == 0 else (valsB, semB)
                nbuf, nsm = (valsB, semB) if g % 2 == 0 else (valsA, semA)
                _wait(buf, sm)
                if g + 1 < GB:
                    _fire(g + 1, nbuf, nsm)
                _scat(g, buf)
            return carry

        lax.fori_loop(0, nblk, blk, 0)

    if compute_deg:
        pltpu.sync_copy(ones_in, ones_b)

    for r in range(2):
        _per_stripe(s, _zero)
        plsc.subcore_barrier()
        if r == 0:
            _run(srcA, dstA, (c, s), NCH_A // GCH // GB)
        else:
            _run(srcB, dstB, (wid,), NCH_B // GCH // GB)
        plsc.subcore_barrier()

        def _wout(off, size, r=r):
            pltpu.sync_copy(acc.at[pl.ds(off, size)],
                            agg_out.at[c, r, pl.ds(off, size)])
            if compute_deg:
                pltpu.sync_copy(dg.at[pl.ds(off, size)],
                                deg_out.at[c, r, pl.ds(off, size)])
        _per_stripe(s, _wout)
        plsc.subcore_barrier()


def _make_segsum(compute_deg):
    mesh = plsc.VectorSubcoreMesh(core_axis_name="c", subcore_axis_name="s",
                                  num_cores=NC, num_subcores=NS)
    out_type = [jax.ShapeDtypeStruct((NC, 2, N, HID), F32),
                jax.ShapeDtypeStruct((NC, 2, N), F32)]
    scratch = [
        pltpu.VMEM_SHARED((N, HID), F32),
        pltpu.VMEM_SHARED((N,), F32),
        pltpu.VMEM((GB, SGW), jnp.int32),
        pltpu.VMEM((GB, SGW), jnp.int32),
        pltpu.VMEM((SGW, HID), F32),
        pltpu.VMEM((SGW, HID), F32),
        pltpu.VMEM((SGW,), F32),
        pltpu.SemaphoreType.DMA,
        pltpu.SemaphoreType.DMA,
    ]
    return pl.kernel(functools.partial(_segsum_body, compute_deg),
                     out_type=out_type, mesh=mesh, scratch_types=scratch,
                     compiler_params=_SC_PARAMS)


def _linkpred_body(zcat, sidx, didx, logit_out,
                   idx_s, idx_d, srA, drA, srB, drB, lbuf,
                   sA1, sA2, sB1, sB2):
    c = lax.axis_index("c")
    s = lax.axis_index("s")
    wid = c * NS + s
    iota16 = lax.iota(jnp.int32, 16)

    def fire(j, sr, dr, s1, s2):
        pltpu.async_copy(zcat.at[idx_s.at[j]], sr, s1)
        pltpu.async_copy(zcat.at[idx_d.at[j]], dr, s2)

    def wait(sr, dr, s1, s2):
        pltpu.make_async_copy(zcat.at[idx_s.at[0]], sr, s1).wait()
        pltpu.make_async_copy(zcat.at[idx_d.at[0]], dr, s2).wait()

    UNR = 5

    def compute(j, sr, dr):
        def egstep(u, carry):
            for v in range(UNR):
                eg = u * UNR + v
                rows = iota16 + eg * 16
                acc0 = jnp.zeros((16,), F32)
                acc1 = jnp.zeros((16,), F32)
                for k in range(0, ODIM, 2):
                    c0 = jnp.full((16,), k, jnp.int32)
                    c1 = jnp.full((16,), k + 1, jnp.int32)
                    acc0 = acc0 + plsc.load_gather(sr, [rows, c0]) * \
                        plsc.load_gather(dr, [rows, c0])
                    acc1 = acc1 + plsc.load_gather(sr, [rows, c1]) * \
                        plsc.load_gather(dr, [rows, c1])
                lbuf[pl.ds(j * SGW + eg * 16, 16)] = acc0 + acc1
            return carry

        lax.fori_loop(0, SGW // 16 // UNR, egstep, 0)

    NSG = NCH_B // GCH
    for g in range(4):
        pltpu.sync_copy(sidx.at[g, wid], idx_s)
        pltpu.sync_copy(didx.at[g, wid], idx_d)
        fire(0, srA, drA, sA1, sA2)

        def pair(p, carry):
            ja = 2 * p
            wait(srA, drA, sA1, sA2)
            fire(ja + 1, srB, drB, sB1, sB2)
            compute(ja, srA, drA)
            wait(srB, drB, sB1, sB2)
            fire(ja + 2, srA, drA, sA1, sA2)
            compute(ja + 1, srB, drB)
            return carry

        lax.fori_loop(0, (NSG - 1) // 2, pair, 0)
        wait(srA, drA, sA1, sA2)
        compute(NSG - 1, srA, drA)
        off = pl.multiple_of((g * NW + wid) * EPT, 8)
        pltpu.sync_copy(lbuf, logit_out.at[pl.ds(off, EPT)])


def _make_linkpred():
    mesh = plsc.VectorSubcoreMesh(core_axis_name="c", subcore_axis_name="s",
                                  num_cores=NC, num_subcores=NS)
    out_type = jax.ShapeDtypeStruct((4 * NW * EPT,), F32)
    scratch = [
        pltpu.VMEM((EPT // SGW, SGW), jnp.int32),
        pltpu.VMEM((EPT // SGW, SGW), jnp.int32),
        pltpu.VMEM((SGW, ODIM), F32),
        pltpu.VMEM((SGW, ODIM), F32),
        pltpu.VMEM((SGW, ODIM), F32),
        pltpu.VMEM((SGW, ODIM), F32),
        pltpu.VMEM((EPT,), F32),
        pltpu.SemaphoreType.DMA,
        pltpu.SemaphoreType.DMA,
        pltpu.SemaphoreType.DMA,
        pltpu.SemaphoreType.DMA,
    ]
    return pl.kernel(_linkpred_body, out_type=out_type, mesh=mesh,
                     scratch_types=scratch,
                     compiler_params=pltpu.CompilerParams(
                         use_tc_tiling_on_sc=False,
                         needs_layout_passes=False))


_RB = 1000


def _proj1_body(x_ref, wn_ref, ws_ref, b1_ref, y1_ref, xs_ref):
    xb = x_ref[...]
    y1_ref[...] = jnp.dot(xb, wn_ref[...], preferred_element_type=F32)
    xs_ref[...] = jnp.dot(xb, ws_ref[...], preferred_element_type=F32) + b1_ref[...]


def _tc_proj1(x, W_neigh1, W_self1, b1):
    nb = N // _RB
    return pl.pallas_call(
        _proj1_body,
        grid=(nb,),
        in_specs=[
            pl.BlockSpec((_RB, 128), lambda i: (i, 0)),
            pl.BlockSpec((128, HID), lambda i: (0, 0)),
            pl.BlockSpec((128, HID), lambda i: (0, 0)),
            pl.BlockSpec((1, HID), lambda i: (0, 0)),
        ],
        out_specs=[
            pl.BlockSpec((_RB, HID), lambda i: (i, 0)),
            pl.BlockSpec((_RB, HID), lambda i: (i, 0)),
        ],
        out_shape=[jax.ShapeDtypeStruct((N, HID), F32),
                   jax.ShapeDtypeStruct((N, HID), F32)],
    )(x, W_neigh1, W_self1, b1.reshape(1, HID))


def _combine3(p0, p1):
    return (p0[0], p0[1] + p1[1], p1[0])


def _mid_body(xs_ref, a0_ref, a1_ref, d0_ref, d1_ref, wn_ref, ws_ref, b2_ref,
              y2_ref, hs_ref, rd_ref):
    aggs = _combine3(a0_ref, a1_ref)
    degs = _combine3(d0_ref, d1_ref)
    xb = xs_ref[...]
    for t in range(T):
        rd = 1.0 / jnp.maximum(degs[t], 1.0)
        rd_ref[t] = rd
        h1 = jnp.maximum(xb + aggs[t] * rd, 0.0)
        y2_ref[t] = jnp.dot(h1, wn_ref[...], preferred_element_type=F32)
        hs_ref[t] = jnp.dot(h1, ws_ref[...], preferred_element_type=F32) + b2_ref[...]


def _tc_mid(xs, ap, dp, W_neigh2, W_self2, b2):
    nb = N // _RB
    pspec = pl.BlockSpec((2, _RB, HID), lambda i: (0, i, 0))
    dspec = pl.BlockSpec((2, _RB, 1), lambda i: (0, i, 0))
    return pl.pallas_call(
        _mid_body,
        grid=(nb,),
        in_specs=[
            pl.BlockSpec((_RB, HID), lambda i: (i, 0)),
            pspec, pspec, dspec, dspec,
            pl.BlockSpec((HID, HID), lambda i: (0, 0)),
            pl.BlockSpec((HID, HID), lambda i: (0, 0)),
            pl.BlockSpec((1, HID), lambda i: (0, 0)),
        ],
        out_specs=[
            pl.BlockSpec((T, _RB, HID), lambda i: (0, i, 0)),
            pl.BlockSpec((T, _RB, HID), lambda i: (0, i, 0)),
            pl.BlockSpec((T, _RB, 1), lambda i: (0, i, 0)),
        ],
        out_shape=[jax.ShapeDtypeStruct((T, N, HID), F32),
                   jax.ShapeDtypeStruct((T, N, HID), F32),
                   jax.ShapeDtypeStruct((T, N, 1), F32)],
    )(xs, ap[0], ap[1], dp[0].reshape(2, N, 1), dp[1].reshape(2, N, 1),
      W_neigh2, W_self2, b2.reshape(1, HID))


def _gru_body(hs_ref, a0_ref, a1_ref, rd_ref,
              wir_ref, wiz_ref, win_ref, whr_ref, whz_ref, whn_ref,
              bir_ref, biz_ref, bin_ref, bhr_ref, bhz_ref, bhn_ref,
              wo_ref, bo_ref, z01_ref, hf_ref):
    aggs = _combine3(a0_ref, a1_ref)
    h = jnp.zeros((_RB, HID), F32)
    for t in range(T):
        xt = hs_ref[t] + aggs[t] * rd_ref[t]
        ir = jnp.dot(xt, wir_ref[...], preferred_element_type=F32) + bir_ref[...]
        iz = jnp.dot(xt, wiz_ref[...], preferred_element_type=F32) + biz_ref[...]
        inn = jnp.dot(xt, win_ref[...], preferred_element_type=F32) + bin_ref[...]
        hr = jnp.dot(h, whr_ref[...], preferred_element_type=F32) + bhr_ref[...]
        hz = jnp.dot(h, whz_ref[...], preferred_element_type=F32) + bhz_ref[...]
        hn = jnp.dot(h, whn_ref[...], preferred_element_type=F32) + bhn_ref[...]
        r = jax.nn.sigmoid(ir + hr)
        z = jax.nn.sigmoid(iz + hz)
        n = jnp.tanh(inn + r * hn)
        h = (1.0 - z) * n + z * h
        if t < 2:
            z01_ref[t] = jnp.dot(h, wo_ref[...], preferred_element_type=F32) + bo_ref[...]
    hf_ref[...] = h


def _tc_gru(hs, ap, rd, W_ih, W_hh, b_ih, b_hh, W_out, b_out):
    nb = N // _RB
    gate_w = [W_ih[:HID].T, W_ih[HID:2 * HID].T, W_ih[2 * HID:].T,
              W_hh[:HID].T, W_hh[HID:2 * HID].T, W_hh[2 * HID:].T]
    gate_b = [b_ih[:HID].reshape(1, HID), b_ih[HID:2 * HID].reshape(1, HID),
              b_ih[2 * HID:].reshape(1, HID), b_hh[:HID].reshape(1, HID),
              b_hh[HID:2 * HID].reshape(1, HID), b_hh[2 * HID:].reshape(1, HID)]
    wspec = pl.BlockSpec((HID, HID), lambda i: (0, 0))
    bspec = pl.BlockSpec((1, HID), lambda i: (0, 0))
    pspec = pl.BlockSpec((2, _RB, HID), lambda i: (0, i, 0))
    return pl.pallas_call(
        _gru_body,
        grid=(nb,),
        in_specs=[
            pl.BlockSpec((T, _RB, HID), lambda i: (0, i, 0)),
            pspec, pspec,
            pl.BlockSpec((T, _RB, 1), lambda i: (0, i, 0)),
            wspec, wspec, wspec, wspec, wspec, wspec,
            bspec, bspec, bspec, bspec, bspec, bspec,
            pl.BlockSpec((HID, ODIM), lambda i: (0, 0)),
            pl.BlockSpec((1, ODIM), lambda i: (0, 0)),
        ],
        out_specs=[
            pl.BlockSpec((2, _RB, ODIM), lambda i: (0, i, 0)),
            pl.BlockSpec((_RB, HID), lambda i: (i, 0)),
        ],
        out_shape=[jax.ShapeDtypeStruct((2, N, ODIM), F32),
                   jax.ShapeDtypeStruct((N, HID), F32)],
    )(hs, ap[0], ap[1], rd, *gate_w, *gate_b, W_out, b_out.reshape(1, ODIM))


_LB = E // 16


def _loss_body(l_ref, o_ref):
    i = pl.program_id(0)
    l = l_ref[...]
    sgn = jnp.where(i < 4, -1.0, 1.0)
    xx = sgn * l
    sp = jnp.maximum(xx, 0.0) + jnp.log1p(jnp.exp(-jnp.abs(xx)))
    ps = jnp.sum(sp) * (1.0 / (4.0 * E))

    @pl.when(i == 0)
    def _init():
        o_ref[...] = jnp.zeros_like(o_ref)

    o_ref[...] += ps


def _tc_loss(logits):
    return pl.pallas_call(
        _loss_body,
        grid=(8,),
        in_specs=[pl.BlockSpec((8, _LB), lambda i: (i, 0))],
        out_specs=pl.BlockSpec((1, 1), lambda i: (0, 0)),
        out_shape=jax.ShapeDtypeStruct((1, 1), F32),
    )(logits)


def _role_split(idx3):
    a = idx3[jnp.array([0, 2])].reshape(2, NS, NCH_A // GCH, SGW)
    b = idx3[1].reshape(NW, NCH_B // GCH, SGW)
    return a, b


def kernel(x, eis, W_self1, W_neigh1, b1, W_self2, W_neigh2, b2,
           W_ih, W_hh, b_ih, b_hh, W_out, b_out):
    eis = eis.astype(jnp.int32)
    src = eis[:, 0, :]
    dst = eis[:, 1, :]
    srcA1, srcB1 = _role_split(src)
    dstA, dstB = _role_split(dst)
    toff = (jnp.arange(T, dtype=jnp.int32) * N)[:, None]
    srcA2, srcB2 = _role_split(src + toff)

    neg_key = jax.random.key(12345)
    rnd = []
    for i in range(T - 1):
        k1, k2 = jax.random.split(jax.random.fold_in(neg_key, i))
        rnd.append((jax.random.randint(k1, (E,), 0, N).astype(jnp.int32),
                    jax.random.randint(k2, (E,), 0, N).astype(jnp.int32)))

    znd = jnp.zeros((N, HID), F32)
    zn = jnp.zeros((N,), F32)
    ones_in = jnp.ones((SGW,), F32)

    y1, xs = _tc_proj1(x, W_neigh1, W_self1, b1)
    agg1, degp = _make_segsum(True)(y1, srcA1, dstA, srcB1, dstB,
                                    znd, zn, ones_in)

    y2, hs, rd = _tc_mid(xs, agg1, degp, W_neigh2, W_self2, b2)
    agg2, _ = _make_segsum(False)(y2.reshape(T * N, HID), srcA2, dstA,
                                  srcB2, dstB, znd, zn, ones_in)

    z01, hfin = _tc_gru(hs, agg2, rd, W_ih, W_hh, b_ih, b_hh, W_out, b_out)

    zcat = z01.reshape(2 * N, ODIM)
    sidx = jnp.stack([src[1], src[2] + N, rnd[0][0], rnd[1][0] + N])
    didx = jnp.stack([dst[1], dst[2] + N, rnd[0][1], rnd[1][1] + N])
    logits = _make_linkpred()(zcat,
                              sidx.reshape(4, NW, NCH_B // GCH, SGW),
                              didx.reshape(4, NW, NCH_B // GCH, SGW))

    loss = _tc_loss(logits.reshape(64, E // 16))[0, 0]
    return (loss, hfin[None])


# speedup vs baseline: 11.8986x; 1.9840x over previous
import functools

import jax
import jax.numpy as jnp
from jax import lax
from jax.experimental import pallas as pl
from jax.experimental.pallas import tpu as pltpu
from jax.experimental.pallas import tpu_sc as plsc

N = 10000
E = 320000
HID = 64
ODIM = 32
ZD = 33
T = 3

NC = 2
NS = 16
NW = NC * NS
CH = 80
NBI = 25
NCH_A = E // NS // CH
NCH_B = E // NW // CH
EPT = E // NW

F32 = jnp.float32

_SC_PARAMS = pltpu.CompilerParams(use_tc_tiling_on_sc=False)


_RS = 632
_RSL = N - _RS * (NS - 1)


def _per_stripe(s, fn):
    @pl.when(s < NS - 1)
    def _a():
        fn(pl.multiple_of(s * _RS, 8), _RS)

    @pl.when(s == NS - 1)
    def _b():
        fn((NS - 1) * _RS, _RSL)


GCH = 5
SGW = GCH * CH
GB = 5


def _segsum_body(compute_deg, table, srcA, dstA, srcB, dstB, znd, zn, ones_in,
                 agg_out, deg_out,
                 acc, dg,
                 idx_s, idx_d, valsA, valsB, ones_b, semA, semB):
    c = lax.axis_index("c")
    s = lax.axis_index("s")
    wid = c * NS + s

    def _zero(off, size):
        pltpu.sync_copy(znd.at[pl.ds(off, size)], acc.at[pl.ds(off, size)])
        if compute_deg:
            pltpu.sync_copy(zn.at[pl.ds(off, size)], dg.at[pl.ds(off, size)])

    def _fire(g, buf, sm):
        pltpu.async_copy(table.at[idx_s.at[g]], buf, sm)

    def _wait(buf, sm):
        pltpu.make_async_copy(table.at[idx_s.at[0]], buf, sm).wait()

    def _scat(g, buf):
        pltpu.sync_copy(buf, acc.at[idx_d.at[g]], add=True)
        if compute_deg:
            pltpu.sync_copy(ones_b, dg.at[idx_d.at[g]], add=True)

    def _run(src_h, dst_h, pre, nblk):
        def blk(b, carry):
            pltpu.sync_copy(src_h.at[pre + (pl.ds(b * GB, GB),)], idx_s)
            pltpu.sync_copy(dst_h.at[pre + (pl.ds(b * GB, GB),)], idx_d)
            _fire(0, valsA, semA)
            for g in range(GB):
                buf, sm = (valsA, semA) if g % 2 == 0 else (valsB, semB)
                nbuf, nsm = (valsB, semB) if g % 2 == 0 else (valsA, semA)
                _wait(buf, sm)
                if g + 1 < GB:
                    _fire(g + 1, nbuf, nsm)
                _scat(g, buf)
            return carry

        lax.fori_loop(0, nblk, blk, 0)

    if compute_deg:
        pltpu.sync_copy(ones_in, ones_b)

    for r in range(2):
        _per_stripe(s, _zero)
        plsc.subcore_barrier()
        if r == 0:
            _run(srcA, dstA, (c, s), NCH_A // GCH // GB)
        else:
            _run(srcB, dstB, (wid,), NCH_B // GCH // GB)
        plsc.subcore_barrier()

        def _wout(off, size, r=r):
            pltpu.sync_copy(acc.at[pl.ds(off, size)],
                            agg_out.at[c, r, pl.ds(off, size)])
            if compute_deg:
                pltpu.sync_copy(dg.at[pl.ds(off, size)],
                                deg_out.at[c, r, pl.ds(off, size)])
        _per_stripe(s, _wout)
        plsc.subcore_barrier()


def _make_segsum(compute_deg):
    mesh = plsc.VectorSubcoreMesh(core_axis_name="c", subcore_axis_name="s",
                                  num_cores=NC, num_subcores=NS)
    out_type = [jax.ShapeDtypeStruct((NC, 2, N, HID), F32),
                jax.ShapeDtypeStruct((NC, 2, N), F32)]
    scratch = [
        pltpu.VMEM_SHARED((N, HID), F32),
        pltpu.VMEM_SHARED((N,), F32),
        pltpu.VMEM((GB, SGW), jnp.int32),
        pltpu.VMEM((GB, SGW), jnp.int32),
        pltpu.VMEM((SGW, HID), F32),
        pltpu.VMEM((SGW, HID), F32),
        pltpu.VMEM((SGW,), F32),
        pltpu.SemaphoreType.DMA,
        pltpu.SemaphoreType.DMA,
    ]
    return pl.kernel(functools.partial(_segsum_body, compute_deg),
                     out_type=out_type, mesh=mesh, scratch_types=scratch,
                     compiler_params=_SC_PARAMS)


def _linkpred_body(zcat, sidx, didx, logit_out,
                   idx_s, idx_d, srA, drA, srB, drB, lbuf,
                   sA1, sA2, sB1, sB2):
    c = lax.axis_index("c")
    s = lax.axis_index("s")
    wid = c * NS + s
    iota16 = lax.iota(jnp.int32, 16)

    def fire(j, sr, dr, s1, s2):
        pltpu.async_copy(zcat.at[idx_s.at[j]], sr, s1)
        pltpu.async_copy(zcat.at[idx_d.at[j]], dr, s2)

    def wait(sr, dr, s1, s2):
        pltpu.make_async_copy(zcat.at[idx_s.at[0]], sr, s1).wait()
        pltpu.make_async_copy(zcat.at[idx_d.at[0]], dr, s2).wait()

    UNR = 5

    def compute(j, sr, dr):
        def egstep(u, carry):
            for v in range(UNR):
                eg = u * UNR + v
                rows = iota16 + eg * 16
                acc0 = jnp.zeros((16,), F32)
                acc1 = jnp.zeros((16,), F32)
                for k in range(0, ODIM, 2):
                    c0 = jnp.full((16,), k, jnp.int32)
                    c1 = jnp.full((16,), k + 1, jnp.int32)
                    acc0 = acc0 + plsc.load_gather(sr, [rows, c0]) * \
                        plsc.load_gather(dr, [rows, c0])
                    acc1 = acc1 + plsc.load_gather(sr, [rows, c1]) * \
                        plsc.load_gather(dr, [rows, c1])
                lbuf[pl.ds(j * SGW + eg * 16, 16)] = acc0 + acc1
            return carry

        lax.fori_loop(0, SGW // 16 // UNR, egstep, 0)

    NSG = NCH_B // GCH
    for g in range(4):
        pltpu.sync_copy(sidx.at[g, wid], idx_s)
        pltpu.sync_copy(didx.at[g, wid], idx_d)
        fire(0, srA, drA, sA1, sA2)

        def pair(p, carry):
            ja = 2 * p
            wait(srA, drA, sA1, sA2)
            fire(ja + 1, srB, drB, sB1, sB2)
            compute(ja, srA, drA)
            wait(srB, drB, sB1, sB2)
            fire(ja + 2, srA, drA, sA1, sA2)
            compute(ja + 1, srB, drB)
            return carry

        lax.fori_loop(0, (NSG - 1) // 2, pair, 0)
        wait(srA, drA, sA1, sA2)
        compute(NSG - 1, srA, drA)
        off = pl.multiple_of((g * NW + wid) * EPT, 8)
        pltpu.sync_copy(lbuf, logit_out.at[pl.ds(off, EPT)])


def _make_linkpred():
    mesh = plsc.VectorSubcoreMesh(core_axis_name="c", subcore_axis_name="s",
                                  num_cores=NC, num_subcores=NS)
    out_type = jax.ShapeDtypeStruct((4 * NW * EPT,), F32)
    scratch = [
        pltpu.VMEM((EPT // SGW, SGW), jnp.int32),
        pltpu.VMEM((EPT // SGW, SGW), jnp.int32),
        pltpu.VMEM((SGW, ZD), F32),
        pltpu.VMEM((SGW, ZD), F32),
        pltpu.VMEM((SGW, ZD), F32),
        pltpu.VMEM((SGW, ZD), F32),
        pltpu.VMEM((EPT,), F32),
        pltpu.SemaphoreType.DMA,
        pltpu.SemaphoreType.DMA,
        pltpu.SemaphoreType.DMA,
        pltpu.SemaphoreType.DMA,
    ]
    return pl.kernel(_linkpred_body, out_type=out_type, mesh=mesh,
                     scratch_types=scratch,
                     compiler_params=pltpu.CompilerParams(
                         use_tc_tiling_on_sc=False,
                         needs_layout_passes=False))


_RB = 1000


def _proj1_body(x_ref, wn_ref, ws_ref, b1_ref, y1_ref, xs_ref):
    xb = x_ref[...]
    y1_ref[...] = jnp.dot(xb, wn_ref[...], preferred_element_type=F32)
    xs_ref[...] = jnp.dot(xb, ws_ref[...], preferred_element_type=F32) + b1_ref[...]


def _tc_proj1(x, W_neigh1, W_self1, b1):
    nb = N // _RB
    return pl.pallas_call(
        _proj1_body,
        grid=(nb,),
        in_specs=[
            pl.BlockSpec((_RB, 128), lambda i: (i, 0)),
            pl.BlockSpec((128, HID), lambda i: (0, 0)),
            pl.BlockSpec((128, HID), lambda i: (0, 0)),
            pl.BlockSpec((1, HID), lambda i: (0, 0)),
        ],
        out_specs=[
            pl.BlockSpec((_RB, HID), lambda i: (i, 0)),
            pl.BlockSpec((_RB, HID), lambda i: (i, 0)),
        ],
        out_shape=[jax.ShapeDtypeStruct((N, HID), F32),
                   jax.ShapeDtypeStruct((N, HID), F32)],
    )(x, W_neigh1, W_self1, b1.reshape(1, HID))


def _combine3(p0, p1):
    return (p0[0], p0[1] + p1[1], p1[0])


def _mid_body(xs_ref, a0_ref, a1_ref, d0_ref, d1_ref, wn_ref, ws_ref, b2_ref,
              y2_ref, hs_ref, rd_ref):
    aggs = _combine3(a0_ref, a1_ref)
    degs = _combine3(d0_ref, d1_ref)
    xb = xs_ref[...]
    for t in range(T):
        rd = 1.0 / jnp.maximum(degs[t], 1.0)
        rd_ref[t] = rd
        h1 = jnp.maximum(xb + aggs[t] * rd, 0.0)
        y2_ref[t] = jnp.dot(h1, wn_ref[...], preferred_element_type=F32)
        hs_ref[t] = jnp.dot(h1, ws_ref[...], preferred_element_type=F32) + b2_ref[...]


def _tc_mid(xs, ap, dp, W_neigh2, W_self2, b2):
    nb = N // _RB
    pspec = pl.BlockSpec((2, _RB, HID), lambda i: (0, i, 0))
    dspec = pl.BlockSpec((2, _RB, 1), lambda i: (0, i, 0))
    return pl.pallas_call(
        _mid_body,
        grid=(nb,),
        in_specs=[
            pl.BlockSpec((_RB, HID), lambda i: (i, 0)),
            pspec, pspec, dspec, dspec,
            pl.BlockSpec((HID, HID), lambda i: (0, 0)),
            pl.BlockSpec((HID, HID), lambda i: (0, 0)),
            pl.BlockSpec((1, HID), lambda i: (0, 0)),
        ],
        out_specs=[
            pl.BlockSpec((T, _RB, HID), lambda i: (0, i, 0)),
            pl.BlockSpec((T, _RB, HID), lambda i: (0, i, 0)),
            pl.BlockSpec((T, _RB, 1), lambda i: (0, i, 0)),
        ],
        out_shape=[jax.ShapeDtypeStruct((T, N, HID), F32),
                   jax.ShapeDtypeStruct((T, N, HID), F32),
                   jax.ShapeDtypeStruct((T, N, 1), F32)],
    )(xs, ap[0], ap[1], dp[0].reshape(2, N, 1), dp[1].reshape(2, N, 1),
      W_neigh2, W_self2, b2.reshape(1, HID))


def _gru_body(hs_ref, a0_ref, a1_ref, rd_ref,
              wir_ref, wiz_ref, win_ref, whr_ref, whz_ref, whn_ref,
              bir_ref, biz_ref, bin_ref, bhr_ref, bhz_ref, bhn_ref,
              wo_ref, bo_ref, z01_ref, hf_ref):
    aggs = _combine3(a0_ref, a1_ref)
    h = jnp.zeros((_RB, HID), F32)
    for t in range(T):
        xt = hs_ref[t] + aggs[t] * rd_ref[t]
        ir = jnp.dot(xt, wir_ref[...], preferred_element_type=F32) + bir_ref[...]
        iz = jnp.dot(xt, wiz_ref[...], preferred_element_type=F32) + biz_ref[...]
        inn = jnp.dot(xt, win_ref[...], preferred_element_type=F32) + bin_ref[...]
        hr = jnp.dot(h, whr_ref[...], preferred_element_type=F32) + bhr_ref[...]
        hz = jnp.dot(h, whz_ref[...], preferred_element_type=F32) + bhz_ref[...]
        hn = jnp.dot(h, whn_ref[...], preferred_element_type=F32) + bhn_ref[...]
        r = jax.nn.sigmoid(ir + hr)
        z = jax.nn.sigmoid(iz + hz)
        n = jnp.tanh(inn + r * hn)
        h = (1.0 - z) * n + z * h
        if t < 2:
            z01_ref[t, :, :ODIM] = jnp.dot(h, wo_ref[...],
                                           preferred_element_type=F32) + bo_ref[...]
            z01_ref[t, :, ODIM:] = jnp.zeros((_RB, ZD - ODIM), F32)
    hf_ref[...] = h


def _tc_gru(hs, ap, rd, W_ih, W_hh, b_ih, b_hh, W_out, b_out):
    nb = N // _RB
    gate_w = [W_ih[:HID].T, W_ih[HID:2 * HID].T, W_ih[2 * HID:].T,
              W_hh[:HID].T, W_hh[HID:2 * HID].T, W_hh[2 * HID:].T]
    gate_b = [b_ih[:HID].reshape(1, HID), b_ih[HID:2 * HID].reshape(1, HID),
              b_ih[2 * HID:].reshape(1, HID), b_hh[:HID].reshape(1, HID),
              b_hh[HID:2 * HID].reshape(1, HID), b_hh[2 * HID:].reshape(1, HID)]
    wspec = pl.BlockSpec((HID, HID), lambda i: (0, 0))
    bspec = pl.BlockSpec((1, HID), lambda i: (0, 0))
    pspec = pl.BlockSpec((2, _RB, HID), lambda i: (0, i, 0))
    return pl.pallas_call(
        _gru_body,
        grid=(nb,),
        in_specs=[
            pl.BlockSpec((T, _RB, HID), lambda i: (0, i, 0)),
            pspec, pspec,
            pl.BlockSpec((T, _RB, 1), lambda i: (0, i, 0)),
            wspec, wspec, wspec, wspec, wspec, wspec,
            bspec, bspec, bspec, bspec, bspec, bspec,
            pl.BlockSpec((HID, ODIM), lambda i: (0, 0)),
            pl.BlockSpec((1, ODIM), lambda i: (0, 0)),
        ],
        out_specs=[
            pl.BlockSpec((2, _RB, ZD), lambda i: (0, i, 0)),
            pl.BlockSpec((_RB, HID), lambda i: (i, 0)),
        ],
        out_shape=[jax.ShapeDtypeStruct((2, N, ZD), F32),
                   jax.ShapeDtypeStruct((N, HID), F32)],
    )(hs, ap[0], ap[1], rd, *gate_w, *gate_b, W_out, b_out.reshape(1, ODIM))


_LB = E // 16


def _loss_body(l_ref, o_ref):
    i = pl.program_id(0)
    l = l_ref[...]
    sgn = jnp.where(i < 4, -1.0, 1.0)
    xx = sgn * l
    sp = jnp.maximum(xx, 0.0) + jnp.log1p(jnp.exp(-jnp.abs(xx)))
    ps = jnp.sum(sp) * (1.0 / (4.0 * E))

    @pl.when(i == 0)
    def _init():
        o_ref[...] = jnp.zeros_like(o_ref)

    o_ref[...] += ps


def _tc_loss(logits):
    return pl.pallas_call(
        _loss_body,
        grid=(8,),
        in_specs=[pl.BlockSpec((8, _LB), lambda i: (i, 0))],
        out_specs=pl.BlockSpec((1, 1), lambda i: (0, 0)),
        out_shape=jax.ShapeDtypeStruct((1, 1), F32),
    )(logits)


def _role_split(idx3):
    a = idx3[jnp.array([0, 2])].reshape(2, NS, NCH_A // GCH, SGW)
    b = idx3[1].reshape(NW, NCH_B // GCH, SGW)
    return a, b


def kernel(x, eis, W_self1, W_neigh1, b1, W_self2, W_neigh2, b2,
           W_ih, W_hh, b_ih, b_hh, W_out, b_out):
    eis = eis.astype(jnp.int32)
    src = eis[:, 0, :]
    dst = eis[:, 1, :]
    srcA1, srcB1 = _role_split(src)
    dstA, dstB = _role_split(dst)
    toff = (jnp.arange(T, dtype=jnp.int32) * N)[:, None]
    srcA2, srcB2 = _role_split(src + toff)

    neg_key = jax.random.key(12345)
    rnd = []
    for i in range(T - 1):
        k1, k2 = jax.random.split(jax.random.fold_in(neg_key, i))
        rnd.append((jax.random.randint(k1, (E,), 0, N).astype(jnp.int32),
                    jax.random.randint(k2, (E,), 0, N).astype(jnp.int32)))

    znd = jnp.zeros((N, HID), F32)
    zn = jnp.zeros((N,), F32)
    ones_in = jnp.ones((SGW,), F32)

    y1, xs = _tc_proj1(x, W_neigh1, W_self1, b1)
    agg1, degp = _make_segsum(True)(y1, srcA1, dstA, srcB1, dstB,
                                    znd, zn, ones_in)

    y2, hs, rd = _tc_mid(xs, agg1, degp, W_neigh2, W_self2, b2)
    agg2, _ = _make_segsum(False)(y2.reshape(T * N, HID), srcA2, dstA,
                                  srcB2, dstB, znd, zn, ones_in)

    z01, hfin = _tc_gru(hs, agg2, rd, W_ih, W_hh, b_ih, b_hh, W_out, b_out)

    zcat = z01.reshape(2 * N, ZD)
    sidx = jnp.stack([src[1], src[2] + N, rnd[0][0], rnd[1][0] + N])
    didx = jnp.stack([dst[1], dst[2] + N, rnd[0][1], rnd[1][1] + N])
    logits = _make_linkpred()(zcat,
                              sidx.reshape(4, NW, NCH_B // GCH, SGW),
                              didx.reshape(4, NW, NCH_B // GCH, SGW))

    loss = _tc_loss(logits.reshape(64, E // 16))[0, 0]
    return (loss, hfin[None])
